# Initial kernel scaffold; baseline (speedup 1.0000x reference)
import jax
import jax.numpy as jnp
from jax.experimental import pallas as pl


def kernel(x, edge_index, edge_attr, pw, dW1, db1, dW2, db2, dW3, db3, dg, dbe, uW1, ub1, uW2, ub2, uW3, ub3, ug, ube):
    raise NotImplementedError("write your pallas kernel here")


# speedup vs baseline: 7.7316x; 7.7316x over previous
import functools

import jax
import jax.numpy as jnp
import numpy as np
from jax import lax
from jax.experimental import pallas as pl
from jax.experimental.pallas import tpu as pltpu
from jax.experimental.pallas import tpu_sc as plsc

N = 10000
D = 128
E = 320000
K = N // 2

NC = 2
NS = 16
CHUNK = 128
E2 = 2 * E
EPAD = ((E2 + NC * NS * CHUNK - 1) // (NC * NS * CHUNK)) * (NC * NS * CHUNK)
NCHUNKS = EPAD // CHUNK
ACC_ROWS = 10240
ZROWS = ACC_ROWS // NS
ROWS_OUT = ACC_ROWS // NS

_SQRT1_2 = np.float32(0.7071067811865476)


def _gelu(h):
    return h * 0.5 * (1.0 + lax.erf(h * _SQRT1_2))


def _prep_body(x_ref, score_ref, xp_ref, sel_ref):
    x = x_ref[...]
    score = score_ref[...]
    bi = lax.bitcast_convert_type(score, jnp.int32)
    u = jnp.where(bi < 0, ~bi, bi ^ jnp.int32(-2147483648)).astype(jnp.uint32)

    def step(i, t):
        bit = lax.shift_left(jnp.uint32(1), jnp.uint32(31) - i.astype(jnp.uint32))
        t2 = t | bit
        cnt = jnp.sum((u >= t2).astype(jnp.int32))
        return jnp.where(cnt >= K, t2, t)

    thr = lax.fori_loop(0, 32, step, jnp.uint32(0))
    sel = (u >= thr).astype(jnp.float32)
    xp_ref[...] = x * (score * sel)
    sel_ref[...] = sel


def _prep(x, score2d):
    return pl.pallas_call(
        _prep_body,
        out_shape=[
            jax.ShapeDtypeStruct((N, D), jnp.float32),
            jax.ShapeDtypeStruct((N, 1), jnp.float32),
        ],
    )(x, score2d)


def _seg_loop(srcs_hbm, dsts_hbm, table_hbm, acc, sidx, didx, rows, sem,
              base, nchunk):
    def body(j, carry):
        off = base + j * CHUNK
        pltpu.sync_copy(srcs_hbm.at[pl.ds(off, CHUNK)], sidx)
        pltpu.sync_copy(dsts_hbm.at[pl.ds(off, CHUNK)], didx)
        pltpu.async_copy(table_hbm.at[sidx], rows, sem).wait()
        pltpu.sync_copy(rows, acc.at[didx], add=True)
        return carry

    lax.fori_loop(0, nchunk, body, 0)


_SC_SCRATCH = lambda: ([
    pltpu.VMEM((CHUNK,), jnp.int32),
    pltpu.VMEM((CHUNK,), jnp.int32),
    pltpu.VMEM((CHUNK, D), jnp.float32),
    pltpu.VMEM_SHARED((ACC_ROWS, D), jnp.float32),
    pltpu.SemaphoreType.DMA,
])


def _make_segsum_split():
    mesh = plsc.VectorSubcoreMesh(core_axis_name="c", subcore_axis_name="s",
                                  num_cores=NC, num_subcores=NS)
    nchunk = NCHUNKS // (NC * NS)

    @functools.partial(
        pl.kernel,
        out_type=jax.ShapeDtypeStruct((NC, ACC_ROWS, D), jnp.float32),
        mesh=mesh,
        scratch_types=_SC_SCRATCH(),
    )
    def k(srcs_hbm, dsts_hbm, table_hbm, zeros_hbm, out_hbm,
          sidx, didx, rows, acc, sem):
        cid = lax.axis_index("c")
        sid = lax.axis_index("s")
        pltpu.sync_copy(zeros_hbm, acc.at[pl.ds(sid * ZROWS, ZROWS)])
        plsc.subcore_barrier()
        base = (cid * NS + sid) * nchunk * CHUNK
        _seg_loop(srcs_hbm, dsts_hbm, table_hbm, acc, sidx, didx, rows, sem,
                  base, nchunk)
        plsc.subcore_barrier()
        pltpu.sync_copy(acc.at[pl.ds(sid * ROWS_OUT, ROWS_OUT)],
                        out_hbm.at[cid].at[pl.ds(sid * ROWS_OUT, ROWS_OUT)])

    return k


def _make_segsum_dual():
    mesh = plsc.VectorSubcoreMesh(core_axis_name="c", subcore_axis_name="s",
                                  num_cores=NC, num_subcores=NS)
    nchunk = NCHUNKS // NS

    @functools.partial(
        pl.kernel,
        out_type=jax.ShapeDtypeStruct((NC, ACC_ROWS, D), jnp.float32),
        mesh=mesh,
        scratch_types=_SC_SCRATCH(),
    )
    def k(srcs_hbm, dsts_hbm, rec_hbm, skip_hbm, zeros_hbm, out_hbm,
          sidx, didx, rows, acc, sem):
        cid = lax.axis_index("c")
        sid = lax.axis_index("s")
        pltpu.sync_copy(zeros_hbm, acc.at[pl.ds(sid * ZROWS, ZROWS)])
        plsc.subcore_barrier()
        base = sid * nchunk * CHUNK

        @pl.when(cid == 0)
        def _():
            _seg_loop(srcs_hbm, dsts_hbm, rec_hbm, acc, sidx, didx, rows, sem,
                      base, nchunk)

        @pl.when(cid == 1)
        def _():
            _seg_loop(srcs_hbm, dsts_hbm, skip_hbm, acc, sidx, didx, rows, sem,
                      base, nchunk)

        plsc.subcore_barrier()
        pltpu.sync_copy(acc.at[pl.ds(sid * ROWS_OUT, ROWS_OUT)],
                        out_hbm.at[cid].at[pl.ds(sid * ROWS_OUT, ROWS_OUT)])

    return k


_ROWS_BLK = 2000


def _ln(h, g, be):
    mu = jnp.mean(h, axis=-1, keepdims=True)
    c = h - mu
    var = jnp.mean(c * c, axis=-1, keepdims=True)
    return c / jnp.sqrt(var + 1e-5) * g + be


def _mlp_down_body(p0_ref, p1_ref, sel_ref, w1_ref, b1_ref, w2_ref, b2_ref,
                   w3_ref, b3_ref, g_ref, be_ref, rec_ref):
    agg = p0_ref[...] + p1_ref[...]
    h = _gelu(jnp.dot(agg, w1_ref[...], preferred_element_type=jnp.float32)
              + b1_ref[...])
    h = _gelu(jnp.dot(h, w2_ref[...], preferred_element_type=jnp.float32)
              + b2_ref[...])
    h = jnp.dot(h, w3_ref[...], preferred_element_type=jnp.float32) + b3_ref[...]
    rec_ref[...] = _ln(h, g_ref[...], be_ref[...]) * sel_ref[...]


def _mlp_up_body(pa_ref, ps_ref, w1a_ref, w1b_ref, b1_ref, w2_ref, b2_ref,
                 w3_ref, b3_ref, g_ref, be_ref, out_ref):
    h = (jnp.dot(pa_ref[...], w1a_ref[...], preferred_element_type=jnp.float32)
         + jnp.dot(ps_ref[...], w1b_ref[...], preferred_element_type=jnp.float32)
         + b1_ref[...])
    h = _gelu(h)
    h = _gelu(jnp.dot(h, w2_ref[...], preferred_element_type=jnp.float32)
              + b2_ref[...])
    h = jnp.dot(h, w3_ref[...], preferred_element_type=jnp.float32) + b3_ref[...]
    out_ref[...] = _ln(h, g_ref[...], be_ref[...])


def _row_spec():
    return pl.BlockSpec((_ROWS_BLK, D), lambda i: (i, 0))


def _full_spec(shape):
    return pl.BlockSpec(shape, lambda i: (0, 0))


def _mlp_down(p0, p1, sel, w1, b1, w2, b2, w3, b3, g, be):
    return pl.pallas_call(
        _mlp_down_body,
        grid=(N // _ROWS_BLK,),
        in_specs=[
            _row_spec(), _row_spec(),
            pl.BlockSpec((_ROWS_BLK, 1), lambda i: (i, 0)),
            _full_spec((D, D)), _full_spec((1, D)),
            _full_spec((D, D)), _full_spec((1, D)),
            _full_spec((D, D)), _full_spec((1, D)),
            _full_spec((1, D)), _full_spec((1, D)),
        ],
        out_specs=_row_spec(),
        out_shape=jax.ShapeDtypeStruct((N, D), jnp.float32),
    )(p0, p1, sel, w1, b1, w2, b2, w3, b3, g, be)


def _mlp_up(pa, ps, w1a, w1b, b1, w2, b2, w3, b3, g, be):
    return pl.pallas_call(
        _mlp_up_body,
        grid=(N // _ROWS_BLK,),
        in_specs=[
            _row_spec(), _row_spec(),
            _full_spec((D, D)), _full_spec((D, D)), _full_spec((1, D)),
            _full_spec((D, D)), _full_spec((1, D)),
            _full_spec((D, D)), _full_spec((1, D)),
            _full_spec((1, D)), _full_spec((1, D)),
        ],
        out_specs=_row_spec(),
        out_shape=jax.ShapeDtypeStruct((N, D), jnp.float32),
    )(pa, ps, w1a, w1b, b1, w2, b2, w3, b3, g, be)


def kernel(x, edge_index, edge_attr, pw, dW1, db1, dW2, db2, dW3, db3, dg, dbe,
           uW1, ub1, uW2, ub2, uW3, ub3, ug, ube):
    del edge_attr
    ei0 = edge_index[0]
    ei1 = edge_index[1]
    pad = EPAD - E2
    srcs = jnp.concatenate([ei0, ei1, jnp.zeros((pad,), jnp.int32)])
    dsts = jnp.concatenate([ei1, ei0, jnp.full((pad,), N, jnp.int32)])
    zeros = jnp.zeros((ZROWS, D), jnp.float32)

    score2d = jnp.tanh((x @ pw) / jnp.linalg.norm(pw))[:, None]
    xp, sel = _prep(x, score2d)

    segsum_split = _make_segsum_split()
    parts1 = segsum_split(srcs, dsts, xp, zeros)

    rec = _mlp_down(parts1[0, :N], parts1[1, :N], sel,
                    dW1, db1.reshape(1, D), dW2, db2.reshape(1, D),
                    dW3, db3.reshape(1, D), dg.reshape(1, D), dbe.reshape(1, D))

    segsum_dual = _make_segsum_dual()
    parts2 = segsum_dual(srcs, dsts, rec, x, zeros)

    out = _mlp_up(parts2[0, :N], parts2[1, :N],
                  uW1[:D], uW1[D:], ub1.reshape(1, D),
                  uW2, ub2.reshape(1, D), uW3, ub3.reshape(1, D),
                  ug.reshape(1, D), ube.reshape(1, D))
    return out
